# baseline (speedup 1.0000x reference)
import jax
import jax.numpy as jnp
import numpy as np
from jax import lax
from jax.experimental import pallas as pl
from jax.experimental.pallas import tpu as pltpu
from jax.experimental.pallas import tpu_sc as plsc

F = 2
N_POINTS = 131072
N_FEAT = 96
HASH_SIZE = 1 << 18
DENSE0 = 33 * 33 * 33
TOTAL = DENSE0 + 11 * HASH_SIZE

P1 = np.int32(2654435761 - (1 << 32))
P2 = np.int32(805459861)
MASK = np.int32(HASH_SIZE - 1)

NC, NS = 2, 16
NW = NC * NS
CHUNK = 512
PPW = N_POINTS // NW
NCHUNK = PPW // CHUNK
NVEC = CHUNK // 16

GRID_DIMS = ((0, 1, 2), (0, 1, 3), (1, 2, 3), (0, 2, 3))

DENSE_DELTAS = tuple((c & 1) + 33 * ((c >> 1) & 1) + 1089 * ((c >> 2) & 1)
                     for c in range(8))


def _body(us, tab, out, u_v, idx_v, dst_v, w8_v, out_v, sem):
    wid = lax.axis_index("s") * NC + lax.axis_index("c")
    iot = lax.iota(jnp.int32, 16)
    iot96 = iot * N_FEAT

    def axis_vals(d, p, res_f):
        pos = u_v[d, pl.ds(p, 16)] * res_f
        i0 = pos.astype(jnp.int32)
        w1 = pos - i0.astype(jnp.float32)
        return i0, w1

    def store_weights(p, wx1, wy1, wz1):
        wx0 = 1.0 - wx1
        wy0 = 1.0 - wy1
        wz0 = 1.0 - wz1
        wxy = (wx0 * wy0, wx1 * wy0, wx0 * wy1, wx1 * wy1)
        for c in range(8):
            wz = wz1 if (c & 4) else wz0
            w8_v[pl.ds(c * CHUNK + p, 16)] = wxy[c & 3] * wz

    def store_idx(p, c, row, goff):
        e0 = row * 2 + goff
        idx_v[pl.ds(c * CHUNK + p, 16)] = e0
        idx_v[pl.ds((8 + c) * CHUNK + p, 16)] = e0 + 1

    def compute_dense(dims, goff):
        def body_i(i, _):
            p = i * 16
            ix, wx1 = axis_vals(dims[0], p, 32.0)
            iy, wy1 = axis_vals(dims[1], p, 32.0)
            iz, wz1 = axis_vals(dims[2], p, 32.0)
            base = ix + iy * 33 + iz * 1089
            for c in range(8):
                store_idx(p, c, base + DENSE_DELTAS[c], goff)
            store_weights(p, wx1, wy1, wz1)
            return 0
        lax.fori_loop(0, NVEC, body_i, 0)

    def compute_hashed(dims, res_f, off, goff):
        def body_i(i, _):
            p = i * 16
            ix, wx1 = axis_vals(dims[0], p, res_f)
            iy, wy1 = axis_vals(dims[1], p, res_f)
            iz, wz1 = axis_vals(dims[2], p, res_f)
            hx = (ix, ix + 1)
            hy0 = iy * P1
            hy = (hy0, hy0 + P1)
            hz0 = iz * P2
            hz = (hz0, hz0 + P2)
            for c in range(8):
                h = hx[c & 1] ^ hy[(c >> 1) & 1] ^ hz[(c >> 2) & 1]
                store_idx(p, c, (h & MASK) + off, goff)
            store_weights(p, wx1, wy1, wz1)
            return 0
        lax.fori_loop(0, NVEC, body_i, 0)

    def drain(col0):
        def body_i(i, _):
            p = i * 16
            acc0 = jnp.zeros((16,), jnp.float32)
            acc1 = jnp.zeros((16,), jnp.float32)
            for c in range(8):
                f0 = dst_v[pl.ds(c * CHUNK + p, 16)]
                f1 = dst_v[pl.ds((8 + c) * CHUNK + p, 16)]
                w = w8_v[pl.ds(c * CHUNK + p, 16)]
                acc0 = acc0 + f0 * w
                acc1 = acc1 + f1 * w
            obase = p * N_FEAT + col0 + iot96
            plsc.store_scatter(out_v, [obase], acc0)
            plsc.store_scatter(out_v, [obase + 1], acc1)
            return 0
        lax.fori_loop(0, NVEC, body_i, 0)

    def chunk_body(ci, _):
        base = wid * PPW + ci * CHUNK
        pltpu.sync_copy(us.at[:, pl.ds(base, CHUNK)], u_v)
        for g in range(4):
            dims = GRID_DIMS[g]
            goff = np.int32(g * 2 * TOTAL)
            compute_dense(dims, goff)
            pltpu.async_copy(tab.at[idx_v], dst_v, sem).wait()
            drain(np.int32(g * 24))

            def lvl_body(l, _, dims=dims, goff=goff, g=g):
                res_f = lax.shift_left(np.int32(32), l).astype(jnp.float32)
                off = np.int32(DENSE0 - HASH_SIZE) + l * np.int32(HASH_SIZE)
                compute_hashed(dims, res_f, off, goff)
                pltpu.async_copy(tab.at[idx_v], dst_v, sem).wait()
                drain(np.int32(g * 24) + 2 * l)
                return 0
            lax.fori_loop(1, 12, lvl_body, 0)
        pltpu.sync_copy(out_v, out.at[pl.ds(base * N_FEAT, CHUNK * N_FEAT)])
        return 0

    lax.fori_loop(0, NCHUNK, chunk_body, 0)


def _encode(us, tab):
    mesh = plsc.VectorSubcoreMesh(core_axis_name="c", subcore_axis_name="s",
                                  num_cores=NC, num_subcores=NS)
    return pl.kernel(
        _body,
        out_type=jax.ShapeDtypeStruct((N_POINTS * N_FEAT,), jnp.float32),
        mesh=mesh,
        scratch_types=[
            pltpu.VMEM((4, CHUNK), jnp.float32),
            pltpu.VMEM((16 * CHUNK,), jnp.int32),
            pltpu.VMEM((16 * CHUNK,), jnp.float32),
            pltpu.VMEM((8 * CHUNK,), jnp.float32),
            pltpu.VMEM((CHUNK * N_FEAT,), jnp.float32),
            pltpu.SemaphoreType.DMA,
        ],
        compiler_params=pltpu.CompilerParams(needs_layout_passes=False,
                                             use_tc_tiling_on_sc=False),
    )(us, tab)


def kernel(coords, xyz_table, xyt_table, yzt_table, xzt_table):
    sp01 = (coords[:, :3] + 1.0) * 0.5
    t01 = (((coords[:, 3:] * 2.0 - 1.0) * 0.9) + 1.0) * 0.5
    us = jnp.concatenate([sp01, t01], axis=1).T
    tab = jnp.concatenate([xyz_table.reshape(-1), xyt_table.reshape(-1),
                           yzt_table.reshape(-1), xzt_table.reshape(-1)])
    flat = _encode(us, tab)
    return flat.reshape(N_POINTS, N_FEAT)


# speedup vs baseline: 1.0420x; 1.0420x over previous
import jax
import jax.numpy as jnp
import numpy as np
from jax import lax
from jax.experimental import pallas as pl
from jax.experimental.pallas import tpu as pltpu
from jax.experimental.pallas import tpu_sc as plsc

F = 2
N_POINTS = 131072
N_FEAT = 96
HASH_SIZE = 1 << 18
DENSE0 = 33 * 33 * 33
TOTAL = DENSE0 + 11 * HASH_SIZE

P1 = np.int32(2654435761 - (1 << 32))
P2 = np.int32(805459861)
MASK = np.int32(HASH_SIZE - 1)

NC, NS = 2, 16
NW = NC * NS
CHUNK = 512
PPW = N_POINTS // NW
NCHUNK = PPW // CHUNK
NVEC = CHUNK // 16

GRID_DIMS = ((0, 1, 2), (0, 1, 3), (1, 2, 3), (0, 2, 3))

DENSE_DELTAS = tuple((c & 1) + 33 * ((c >> 1) & 1) + 1089 * ((c >> 2) & 1)
                     for c in range(8))


def _body(us, t0, t1, t2, t3, out, u_v, idx_v, dst_v, w8_v, out_v, sem):
    tables = (t0, t1, t2, t3)
    wid = lax.axis_index("s") * NC + lax.axis_index("c")
    iot = lax.iota(jnp.int32, 16)
    iot96 = iot * N_FEAT

    def axis_vals(d, p, res_f):
        pos = u_v[d, pl.ds(p, 16)] * res_f
        i0 = pos.astype(jnp.int32)
        w1 = pos - i0.astype(jnp.float32)
        return i0, w1

    def store_weights(p, wx1, wy1, wz1):
        wx0 = 1.0 - wx1
        wy0 = 1.0 - wy1
        wz0 = 1.0 - wz1
        wxy = (wx0 * wy0, wx1 * wy0, wx0 * wy1, wx1 * wy1)
        for c in range(8):
            wz = wz1 if (c & 4) else wz0
            w8_v[pl.ds(c * CHUNK + p, 16)] = wxy[c & 3] * wz

    def store_idx(p, c, row):
        e0 = row * 2
        idx_v[pl.ds(c * CHUNK + p, 16)] = e0
        idx_v[pl.ds((8 + c) * CHUNK + p, 16)] = e0 + 1

    def compute_dense(dims):
        def body_i(i, _):
            p = i * 16
            ix, wx1 = axis_vals(dims[0], p, 32.0)
            iy, wy1 = axis_vals(dims[1], p, 32.0)
            iz, wz1 = axis_vals(dims[2], p, 32.0)
            base = ix + iy * 33 + iz * 1089
            for c in range(8):
                store_idx(p, c, base + DENSE_DELTAS[c])
            store_weights(p, wx1, wy1, wz1)
            return 0
        lax.fori_loop(0, NVEC, body_i, 0)

    def compute_hashed(dims, res_f, off):
        def body_i(i, _):
            p = i * 16
            ix, wx1 = axis_vals(dims[0], p, res_f)
            iy, wy1 = axis_vals(dims[1], p, res_f)
            iz, wz1 = axis_vals(dims[2], p, res_f)
            hx = (ix, ix + 1)
            hy0 = iy * P1
            hy = (hy0, hy0 + P1)
            hz0 = iz * P2
            hz = (hz0, hz0 + P2)
            for c in range(8):
                h = hx[c & 1] ^ hy[(c >> 1) & 1] ^ hz[(c >> 2) & 1]
                store_idx(p, c, (h & MASK) + off)
            store_weights(p, wx1, wy1, wz1)
            return 0
        lax.fori_loop(0, NVEC, body_i, 0)

    def drain(col0):
        def body_i(i, _):
            p = i * 16
            acc0 = jnp.zeros((16,), jnp.float32)
            acc1 = jnp.zeros((16,), jnp.float32)
            for c in range(8):
                f0 = dst_v[pl.ds(c * CHUNK + p, 16)]
                f1 = dst_v[pl.ds((8 + c) * CHUNK + p, 16)]
                w = w8_v[pl.ds(c * CHUNK + p, 16)]
                acc0 = acc0 + f0 * w
                acc1 = acc1 + f1 * w
            obase = p * N_FEAT + col0 + iot96
            plsc.store_scatter(out_v, [obase], acc0)
            plsc.store_scatter(out_v, [obase + 1], acc1)
            return 0
        lax.fori_loop(0, NVEC, body_i, 0)

    def chunk_body(ci, _):
        base = wid * PPW + ci * CHUNK
        pltpu.sync_copy(us.at[:, pl.ds(base, CHUNK)], u_v)
        for g in range(4):
            dims = GRID_DIMS[g]
            table = tables[g]
            compute_dense(dims)
            pltpu.async_copy(table.at[idx_v], dst_v, sem).wait()
            drain(np.int32(g * 24))

            def lvl_body(l, _, table=table, dims=dims, g=g):
                res_f = lax.shift_left(np.int32(32), l).astype(jnp.float32)
                off = np.int32(DENSE0 - HASH_SIZE) + l * np.int32(HASH_SIZE)
                compute_hashed(dims, res_f, off)
                pltpu.async_copy(table.at[idx_v], dst_v, sem).wait()
                drain(np.int32(g * 24) + 2 * l)
                return 0
            lax.fori_loop(1, 12, lvl_body, 0)
        pltpu.sync_copy(out_v, out.at[pl.ds(base * N_FEAT, CHUNK * N_FEAT)])
        return 0

    lax.fori_loop(0, NCHUNK, chunk_body, 0)


def _encode(us, t0, t1, t2, t3):
    mesh = plsc.VectorSubcoreMesh(core_axis_name="c", subcore_axis_name="s",
                                  num_cores=NC, num_subcores=NS)
    return pl.kernel(
        _body,
        out_type=jax.ShapeDtypeStruct((N_POINTS * N_FEAT,), jnp.float32),
        mesh=mesh,
        scratch_types=[
            pltpu.VMEM((4, CHUNK), jnp.float32),
            pltpu.VMEM((16 * CHUNK,), jnp.int32),
            pltpu.VMEM((16 * CHUNK,), jnp.float32),
            pltpu.VMEM((8 * CHUNK,), jnp.float32),
            pltpu.VMEM((CHUNK * N_FEAT,), jnp.float32),
            pltpu.SemaphoreType.DMA,
        ],
        compiler_params=pltpu.CompilerParams(needs_layout_passes=False,
                                             use_tc_tiling_on_sc=False),
    )(us, t0, t1, t2, t3)


def kernel(coords, xyz_table, xyt_table, yzt_table, xzt_table):
    sp01 = (coords[:, :3] + 1.0) * 0.5
    t01 = (((coords[:, 3:] * 2.0 - 1.0) * 0.9) + 1.0) * 0.5
    us = jnp.concatenate([sp01, t01], axis=1).T
    flats = lax.optimization_barrier(
        (xyz_table.reshape(-1), xyt_table.reshape(-1),
         yzt_table.reshape(-1), xzt_table.reshape(-1)))
    flat = _encode(us, *flats)
    return flat.reshape(N_POINTS, N_FEAT)


# speedup vs baseline: 3.5164x; 3.3746x over previous
import jax
import jax.numpy as jnp
import numpy as np
from jax import lax
from jax.experimental import pallas as pl
from jax.experimental.pallas import tpu as pltpu
from jax.experimental.pallas import tpu_sc as plsc

F = 2
N_POINTS = 131072
N_FEAT = 96
HASH_SIZE = 1 << 18
DENSE0 = 33 * 33 * 33
TOTAL = DENSE0 + 11 * HASH_SIZE

P1 = np.int32(2654435761 - (1 << 32))
P2 = np.int32(805459861)
MASK = np.int32(HASH_SIZE - 1)

NC, NS = 2, 16
NW = NC * NS
CHUNK = 512
PPW = N_POINTS // NW
NCHUNK = PPW // CHUNK
NVEC = CHUNK // 16

GRID_DIMS = ((0, 1, 2), (0, 1, 3), (1, 2, 3), (0, 2, 3))

DENSE_DELTAS = tuple((c & 1) + 33 * ((c >> 1) & 1) + 1089 * ((c >> 2) & 1)
                     for c in range(8))


def _body(us, t0a, t0b, t1a, t1b, t2a, t2b, t3a, t3b, out,
          u_v, idx_v, dst_v, w8_v, out_v, sem):
    tables = ((t0a, t0b), (t1a, t1b), (t2a, t2b), (t3a, t3b))
    wid = lax.axis_index("s") * NC + lax.axis_index("c")
    iot = lax.iota(jnp.int32, 16)
    iot96 = iot * N_FEAT

    def axis_vals(d, p, res_f):
        pos = u_v[d, pl.ds(p, 16)] * res_f
        i0 = pos.astype(jnp.int32)
        w1 = pos - i0.astype(jnp.float32)
        return i0, w1

    def store_weights(p, wx1, wy1, wz1):
        wx0 = 1.0 - wx1
        wy0 = 1.0 - wy1
        wz0 = 1.0 - wz1
        wxy = (wx0 * wy0, wx1 * wy0, wx0 * wy1, wx1 * wy1)
        for c in range(8):
            wz = wz1 if (c & 4) else wz0
            w8_v[pl.ds(c * CHUNK + p, 16)] = wxy[c & 3] * wz

    def store_idx(p, c, row):
        idx_v[pl.ds(c * CHUNK + p, 16)] = row

    def compute_dense(dims):
        def body_i(i, _):
            p = i * 16
            ix, wx1 = axis_vals(dims[0], p, 32.0)
            iy, wy1 = axis_vals(dims[1], p, 32.0)
            iz, wz1 = axis_vals(dims[2], p, 32.0)
            base = ix + iy * 33 + iz * 1089
            for c in range(8):
                store_idx(p, c, base + DENSE_DELTAS[c])
            store_weights(p, wx1, wy1, wz1)
            return 0
        lax.fori_loop(0, NVEC, body_i, 0)

    def compute_hashed(dims, res_f, off):
        def body_i(i, _):
            p = i * 16
            ix, wx1 = axis_vals(dims[0], p, res_f)
            iy, wy1 = axis_vals(dims[1], p, res_f)
            iz, wz1 = axis_vals(dims[2], p, res_f)
            hx = (ix, ix + 1)
            hy0 = iy * P1
            hy = (hy0, hy0 + P1)
            hz0 = iz * P2
            hz = (hz0, hz0 + P2)
            for c in range(8):
                h = hx[c & 1] ^ hy[(c >> 1) & 1] ^ hz[(c >> 2) & 1]
                store_idx(p, c, (h & MASK) + off)
            store_weights(p, wx1, wy1, wz1)
            return 0
        lax.fori_loop(0, NVEC, body_i, 0)

    def drain(col0):
        def body_i(i, _):
            p = i * 16
            acc0 = jnp.zeros((16,), jnp.float32)
            acc1 = jnp.zeros((16,), jnp.float32)
            for c in range(8):
                f0 = dst_v[pl.ds(c * CHUNK + p, 16)]
                f1 = dst_v[pl.ds((8 * CHUNK) + c * CHUNK + p, 16)]
                w = w8_v[pl.ds(c * CHUNK + p, 16)]
                acc0 = acc0 + f0 * w
                acc1 = acc1 + f1 * w
            obase = p * N_FEAT + col0 + iot96
            plsc.store_scatter(out_v, [obase], acc0)
            plsc.store_scatter(out_v, [obase + 1], acc1)
            return 0
        lax.fori_loop(0, NVEC, body_i, 0)

    def chunk_body(ci, _):
        base = wid * PPW + ci * CHUNK
        pltpu.sync_copy(us.at[:, pl.ds(base, CHUNK)], u_v)
        for g in range(4):
            dims = GRID_DIMS[g]
            ta, tb = tables[g]

            def fire_and_drain(col0, ta=ta, tb=tb):
                ca = pltpu.async_copy(ta.at[idx_v],
                                      dst_v.at[pl.ds(0, 8 * CHUNK)], sem)
                cb = pltpu.async_copy(tb.at[idx_v],
                                      dst_v.at[pl.ds(8 * CHUNK, 8 * CHUNK)],
                                      sem)
                ca.wait()
                cb.wait()
                drain(col0)

            compute_dense(dims)
            fire_and_drain(np.int32(g * 24))

            def lvl_body(l, _, dims=dims, g=g, fd=fire_and_drain):
                res_f = lax.shift_left(np.int32(32), l).astype(jnp.float32)
                off = np.int32(DENSE0 - HASH_SIZE) + l * np.int32(HASH_SIZE)
                compute_hashed(dims, res_f, off)
                fd(np.int32(g * 24) + 2 * l)
                return 0
            lax.fori_loop(1, 12, lvl_body, 0)
        pltpu.sync_copy(out_v, out.at[pl.ds(base * N_FEAT, CHUNK * N_FEAT)])
        return 0

    lax.fori_loop(0, NCHUNK, chunk_body, 0)


def _encode(us, *tcols):
    mesh = plsc.VectorSubcoreMesh(core_axis_name="c", subcore_axis_name="s",
                                  num_cores=NC, num_subcores=NS)
    return pl.kernel(
        _body,
        out_type=jax.ShapeDtypeStruct((N_POINTS * N_FEAT,), jnp.float32),
        mesh=mesh,
        scratch_types=[
            pltpu.VMEM((4, CHUNK), jnp.float32),
            pltpu.VMEM((8 * CHUNK,), jnp.int32),
            pltpu.VMEM((16 * CHUNK,), jnp.float32),
            pltpu.VMEM((8 * CHUNK,), jnp.float32),
            pltpu.VMEM((CHUNK * N_FEAT,), jnp.float32),
            pltpu.SemaphoreType.DMA,
        ],
        compiler_params=pltpu.CompilerParams(needs_layout_passes=False,
                                             use_tc_tiling_on_sc=False),
    )(us, *tcols)


def kernel(coords, xyz_table, xyt_table, yzt_table, xzt_table):
    sp01 = (coords[:, :3] + 1.0) * 0.5
    t01 = (((coords[:, 3:] * 2.0 - 1.0) * 0.9) + 1.0) * 0.5
    us = jnp.concatenate([sp01, t01], axis=1).T
    cols = []
    for t in (xyz_table, xyt_table, yzt_table, xzt_table):
        cols.append(t[:, 0])
        cols.append(t[:, 1])
    flat = _encode(us, *cols)
    return flat.reshape(N_POINTS, N_FEAT)


# speedup vs baseline: 4.1235x; 1.1726x over previous
import jax
import jax.numpy as jnp
import numpy as np
from jax import lax
from jax.experimental import pallas as pl
from jax.experimental.pallas import tpu as pltpu
from jax.experimental.pallas import tpu_sc as plsc

F = 2
N_POINTS = 131072
N_FEAT = 96
HASH_SIZE = 1 << 18
DENSE0 = 33 * 33 * 33
TOTAL = DENSE0 + 11 * HASH_SIZE

P1 = np.int32(2654435761 - (1 << 32))
P2 = np.int32(805459861)
MASK = np.int32(HASH_SIZE - 1)

NC, NS = 2, 16
NW = NC * NS
CHUNK = 512
PPW = N_POINTS // NW
NCHUNK = PPW // CHUNK
NVEC = CHUNK // 16
HALF = CHUNK // 2
NVH = HALF // 16

GRID_DIMS = ((0, 1, 2), (0, 1, 3), (1, 2, 3), (0, 2, 3))

DENSE_DELTAS = tuple((c & 1) + 33 * ((c >> 1) & 1) + 1089 * ((c >> 2) & 1)
                     for c in range(8))


def _body(us, t0a, t0b, t1a, t1b, t2a, t2b, t3a, t3b, out,
          u_v, idx_v, dst_v, w8_v, out_v, sem_a, sem_b):
    tables = ((t0a, t0b), (t1a, t1b), (t2a, t2b), (t3a, t3b))
    wid = lax.axis_index("s") * NC + lax.axis_index("c")
    iot = lax.iota(jnp.int32, 16)
    iot96 = iot * N_FEAT

    def axis_vals(d, p, res_f):
        pos = u_v[d, pl.ds(p, 16)] * res_f
        i0 = pos.astype(jnp.int32)
        w1 = pos - i0.astype(jnp.float32)
        return i0, w1

    def store_weights(p, wx1, wy1, wz1):
        wx0 = 1.0 - wx1
        wy0 = 1.0 - wy1
        wz0 = 1.0 - wz1
        wxy = (wx0 * wy0, wx1 * wy0, wx0 * wy1, wx1 * wy1)
        for c in range(8):
            wz = wz1 if (c & 4) else wz0
            w8_v[pl.ds(c * CHUNK + p, 16)] = wxy[c & 3] * wz

    def store_idx(half, q, c, row):
        idx_v[pl.ds(half * (8 * HALF) + c * HALF + q, 16)] = row

    def compute_dense(dims, half):
        def body_i(i, _):
            q = i * 16
            p = half * HALF + q
            ix, wx1 = axis_vals(dims[0], p, 32.0)
            iy, wy1 = axis_vals(dims[1], p, 32.0)
            iz, wz1 = axis_vals(dims[2], p, 32.0)
            base = ix + iy * 33 + iz * 1089
            for c in range(8):
                store_idx(half, q, c, base + DENSE_DELTAS[c])
            store_weights(p, wx1, wy1, wz1)
            return 0
        lax.fori_loop(0, NVH, body_i, 0)

    def compute_hashed(dims, res_f, off, half):
        def body_i(i, _):
            q = i * 16
            p = half * HALF + q
            ix, wx1 = axis_vals(dims[0], p, res_f)
            iy, wy1 = axis_vals(dims[1], p, res_f)
            iz, wz1 = axis_vals(dims[2], p, res_f)
            hx = (ix, ix + 1)
            hy0 = iy * P1
            hy = (hy0, hy0 + P1)
            hz0 = iz * P2
            hz = (hz0, hz0 + P2)
            for c in range(8):
                h = hx[c & 1] ^ hy[(c >> 1) & 1] ^ hz[(c >> 2) & 1]
                store_idx(half, q, c, (h & MASK) + off)
            store_weights(p, wx1, wy1, wz1)
            return 0
        lax.fori_loop(0, NVH, body_i, 0)

    def fire(half, ta, tb, sem):
        ib = half * (8 * HALF)
        db = half * (16 * HALF)
        idx = idx_v.at[pl.ds(ib, 8 * HALF)]
        ca = pltpu.async_copy(ta.at[idx], dst_v.at[pl.ds(db, 8 * HALF)], sem)
        cb = pltpu.async_copy(tb.at[idx],
                              dst_v.at[pl.ds(db + 8 * HALF, 8 * HALF)], sem)
        return ca, cb

    def drain(col0, half):
        db = half * (16 * HALF)
        def body_i(i, _):
            q = i * 16
            p = half * HALF + q
            acc0 = jnp.zeros((16,), jnp.float32)
            acc1 = jnp.zeros((16,), jnp.float32)
            for c in range(8):
                f0 = dst_v[pl.ds(db + c * HALF + q, 16)]
                f1 = dst_v[pl.ds(db + (8 * HALF) + c * HALF + q, 16)]
                w = w8_v[pl.ds(c * CHUNK + p, 16)]
                acc0 = acc0 + f0 * w
                acc1 = acc1 + f1 * w
            obase = p * N_FEAT + col0 + iot96
            plsc.store_scatter(out_v, [obase], acc0)
            plsc.store_scatter(out_v, [obase + 1], acc1)
            return 0
        lax.fori_loop(0, NVH, body_i, 0)

    def chunk_body(ci, _):
        base = wid * PPW + ci * CHUNK
        pltpu.sync_copy(us.at[:, pl.ds(base, CHUNK)], u_v)
        for g in range(4):
            dims = GRID_DIMS[g]
            ta, tb = tables[g]

            def run_stage(col0, compute, ta=ta, tb=tb):
                compute(0)
                c0a, c0b = fire(0, ta, tb, sem_a)
                compute(1)
                c1a, c1b = fire(1, ta, tb, sem_b)
                c0a.wait()
                c0b.wait()
                drain(col0, 0)
                c1a.wait()
                c1b.wait()
                drain(col0, 1)

            run_stage(np.int32(g * 24), lambda h, dims=dims: compute_dense(dims, h))

            def lvl_body(l, _, dims=dims, g=g, rs=run_stage):
                res_f = lax.shift_left(np.int32(32), l).astype(jnp.float32)
                off = np.int32(DENSE0 - HASH_SIZE) + l * np.int32(HASH_SIZE)
                rs(np.int32(g * 24) + 2 * l,
                   lambda h: compute_hashed(dims, res_f, off, h))
                return 0
            lax.fori_loop(1, 12, lvl_body, 0)
        pltpu.sync_copy(out_v, out.at[pl.ds(base * N_FEAT, CHUNK * N_FEAT)])
        return 0

    lax.fori_loop(0, NCHUNK, chunk_body, 0)


def _encode(us, *tcols):
    mesh = plsc.VectorSubcoreMesh(core_axis_name="c", subcore_axis_name="s",
                                  num_cores=NC, num_subcores=NS)
    return pl.kernel(
        _body,
        out_type=jax.ShapeDtypeStruct((N_POINTS * N_FEAT,), jnp.float32),
        mesh=mesh,
        scratch_types=[
            pltpu.VMEM((4, CHUNK), jnp.float32),
            pltpu.VMEM((8 * CHUNK,), jnp.int32),
            pltpu.VMEM((16 * CHUNK,), jnp.float32),
            pltpu.VMEM((8 * CHUNK,), jnp.float32),
            pltpu.VMEM((CHUNK * N_FEAT,), jnp.float32),
            pltpu.SemaphoreType.DMA,
            pltpu.SemaphoreType.DMA,
        ],
        compiler_params=pltpu.CompilerParams(needs_layout_passes=False,
                                             use_tc_tiling_on_sc=False),
    )(us, *tcols)


def kernel(coords, xyz_table, xyt_table, yzt_table, xzt_table):
    sp01 = (coords[:, :3] + 1.0) * 0.5
    t01 = (((coords[:, 3:] * 2.0 - 1.0) * 0.9) + 1.0) * 0.5
    us = jnp.concatenate([sp01, t01], axis=1).T
    cols = []
    for t in (xyz_table, xyt_table, yzt_table, xzt_table):
        cols.append(t[:, 0])
        cols.append(t[:, 1])
    flat = _encode(us, *cols)
    return flat.reshape(N_POINTS, N_FEAT)


# speedup vs baseline: 4.1521x; 1.0070x over previous
import jax
import jax.numpy as jnp
import numpy as np
from jax import lax
from jax.experimental import pallas as pl
from jax.experimental.pallas import tpu as pltpu
from jax.experimental.pallas import tpu_sc as plsc

F = 2
N_POINTS = 131072
N_FEAT = 96
HASH_SIZE = 1 << 18
DENSE0 = 33 * 33 * 33
TOTAL = DENSE0 + 11 * HASH_SIZE

P1 = np.int32(2654435761 - (1 << 32))
P2 = np.int32(805459861)
MASK = np.int32(HASH_SIZE - 1)

NC, NS = 2, 16
NW = NC * NS
CHUNK = 512
PPW = N_POINTS // NW
NCHUNK = PPW // CHUNK
NVEC = CHUNK // 16
HALF = CHUNK // 2
NVH = HALF // 16

GRID_DIMS = ((0, 1, 2), (0, 1, 3), (1, 2, 3), (0, 2, 3))

DENSE_DELTAS = tuple((c & 1) + 33 * ((c >> 1) & 1) + 1089 * ((c >> 2) & 1)
                     for c in range(8))


def _body(us, t0a, t0b, t1a, t1b, t2a, t2b, t3a, t3b, out,
          u_v, idx_v, dst_v, w8_v, out_v, sem_a, sem_b):
    tables = ((t0a, t0b), (t1a, t1b), (t2a, t2b), (t3a, t3b))
    wid = lax.axis_index("s") * NC + lax.axis_index("c")
    iot = lax.iota(jnp.int32, 16)
    iot96 = iot * N_FEAT

    def axis_vals(d, p, res_f):
        pos = u_v[d, pl.ds(p, 16)] * res_f
        i0 = pos.astype(jnp.int32)
        w1 = pos - i0.astype(jnp.float32)
        return i0, w1

    def store_weights(par, p, wx1, wy1, wz1):
        wx0 = 1.0 - wx1
        wy0 = 1.0 - wy1
        wz0 = 1.0 - wz1
        wxy = (wx0 * wy0, wx1 * wy0, wx0 * wy1, wx1 * wy1)
        for c in range(8):
            wz = wz1 if (c & 4) else wz0
            w8_v[par, pl.ds(c * CHUNK + p, 16)] = wxy[c & 3] * wz

    SEMS = (sem_a, sem_b)

    def store_idx(par, p, c, row):
        idx_v[par, pl.ds(c * CHUNK + p, 16)] = row

    def compute_dense(dims, par):
        def body_i(i, _):
            p = i * 16
            ix, wx1 = axis_vals(dims[0], p, 32.0)
            iy, wy1 = axis_vals(dims[1], p, 32.0)
            iz, wz1 = axis_vals(dims[2], p, 32.0)
            base = ix + iy * 33 + iz * 1089
            for c in range(8):
                store_idx(par, p, c, base + DENSE_DELTAS[c])
            store_weights(par, p, wx1, wy1, wz1)
            return 0
        lax.fori_loop(0, NVEC, body_i, 0)

    def compute_hashed(dims, res_f, off, par):
        def body_i(i, _):
            p = i * 16
            ix, wx1 = axis_vals(dims[0], p, res_f)
            iy, wy1 = axis_vals(dims[1], p, res_f)
            iz, wz1 = axis_vals(dims[2], p, res_f)
            hx = (ix, ix + 1)
            hy0 = iy * P1
            hy = (hy0, hy0 + P1)
            hz0 = iz * P2
            hz = (hz0, hz0 + P2)
            for c in range(8):
                h = hx[c & 1] ^ hy[(c >> 1) & 1] ^ hz[(c >> 2) & 1]
                store_idx(par, p, c, (h & MASK) + off)
            store_weights(par, p, wx1, wy1, wz1)
            return 0
        lax.fori_loop(0, NVEC, body_i, 0)

    def fire(par, ta, tb):
        idx = idx_v.at[par]
        pltpu.async_copy(ta.at[idx], dst_v.at[par, pl.ds(0, 8 * CHUNK)],
                         SEMS[par])
        pltpu.async_copy(tb.at[idx],
                         dst_v.at[par, pl.ds(8 * CHUNK, 8 * CHUNK)],
                         SEMS[par])

    def wait(par, ta, tb):
        idx = idx_v.at[par]
        pltpu.make_async_copy(ta.at[idx],
                              dst_v.at[par, pl.ds(0, 8 * CHUNK)],
                              SEMS[par]).wait()
        pltpu.make_async_copy(tb.at[idx],
                              dst_v.at[par, pl.ds(8 * CHUNK, 8 * CHUNK)],
                              SEMS[par]).wait()

    def drain(col0, par):
        def body_i(i, _):
            p = i * 16
            acc0 = jnp.zeros((16,), jnp.float32)
            acc1 = jnp.zeros((16,), jnp.float32)
            for c in range(8):
                f0 = dst_v[par, pl.ds(c * CHUNK + p, 16)]
                f1 = dst_v[par, pl.ds((8 * CHUNK) + c * CHUNK + p, 16)]
                w = w8_v[par, pl.ds(c * CHUNK + p, 16)]
                acc0 = acc0 + f0 * w
                acc1 = acc1 + f1 * w
            obase = p * N_FEAT + col0 + iot96
            plsc.store_scatter(out_v, [obase], acc0)
            plsc.store_scatter(out_v, [obase + 1], acc1)
            return 0
        lax.fori_loop(0, NVEC, body_i, 0)

    def hashed_stage(g, l, par):
        res_f = lax.shift_left(np.int32(32), l).astype(jnp.float32)
        off = np.int32(DENSE0 - HASH_SIZE) + l * np.int32(HASH_SIZE)
        compute_hashed(GRID_DIMS[g], res_f, off, par)

    def chunk_body(ci, _):
        base = wid * PPW + ci * CHUNK
        pltpu.sync_copy(us.at[:, pl.ds(base, CHUNK)], u_v)
        for g in range(4):
            ta, tb = tables[g]
            ta_p, tb_p = tables[g - 1] if g else tables[0]

            compute_dense(GRID_DIMS[g], 0)
            fire(0, ta, tb)
            if g:
                wait(1, ta_p, tb_p)
                drain(np.int32((g - 1) * 24 + 22), 1)

            def pair_body(k, _, g=g, ta=ta, tb=tb):
                l1 = 2 * k + 1
                hashed_stage(g, l1, 1)
                fire(1, ta, tb)
                wait(0, ta, tb)
                drain(np.int32(g * 24) + 2 * (l1 - 1), 0)
                hashed_stage(g, l1 + 1, 0)
                fire(0, ta, tb)
                wait(1, ta, tb)
                drain(np.int32(g * 24) + 2 * l1, 1)
                return 0
            lax.fori_loop(0, 5, pair_body, 0)

            hashed_stage(g, np.int32(11), 1)
            fire(1, ta, tb)
            wait(0, ta, tb)
            drain(np.int32(g * 24 + 20), 0)

        wait(1, tables[3][0], tables[3][1])
        drain(np.int32(94), 1)
        pltpu.sync_copy(out_v, out.at[pl.ds(base * N_FEAT, CHUNK * N_FEAT)])
        return 0

    lax.fori_loop(0, NCHUNK, chunk_body, 0)


def _encode(us, *tcols):
    mesh = plsc.VectorSubcoreMesh(core_axis_name="c", subcore_axis_name="s",
                                  num_cores=NC, num_subcores=NS)
    return pl.kernel(
        _body,
        out_type=jax.ShapeDtypeStruct((N_POINTS * N_FEAT,), jnp.float32),
        mesh=mesh,
        scratch_types=[
            pltpu.VMEM((4, CHUNK), jnp.float32),
            pltpu.VMEM((2, 8 * CHUNK), jnp.int32),
            pltpu.VMEM((2, 16 * CHUNK), jnp.float32),
            pltpu.VMEM((2, 8 * CHUNK), jnp.float32),
            pltpu.VMEM((CHUNK * N_FEAT,), jnp.float32),
            pltpu.SemaphoreType.DMA,
            pltpu.SemaphoreType.DMA,
        ],
        compiler_params=pltpu.CompilerParams(needs_layout_passes=False,
                                             use_tc_tiling_on_sc=False),
    )(us, *tcols)


def kernel(coords, xyz_table, xyt_table, yzt_table, xzt_table):
    sp01 = (coords[:, :3] + 1.0) * 0.5
    t01 = (((coords[:, 3:] * 2.0 - 1.0) * 0.9) + 1.0) * 0.5
    us = jnp.concatenate([sp01, t01], axis=1).T
    cols = []
    for t in (xyz_table, xyt_table, yzt_table, xzt_table):
        cols.append(t[:, 0])
        cols.append(t[:, 1])
    flat = _encode(us, *cols)
    return flat.reshape(N_POINTS, N_FEAT)
